# baseline (speedup 1.0000x reference)
import functools

import jax
import jax.numpy as jnp
from jax import lax
from jax.experimental import pallas as pl
from jax.experimental.pallas import tpu as pltpu
from jax.experimental.pallas import tpu_sc as plsc

H = 128
NC, NS = 2, 16
NW = NC * NS

PAR_P = 5120
L2_OFF, L1_OFF, L0_OFF = 0, 4096, 4608
PC = 16384
P3 = 3 * PC
XPF_G = P3 + 4096 + 512

_mesh = plsc.VectorSubcoreMesh(
    core_axis_name="c", subcore_axis_name="s", num_cores=NC, num_subcores=NS)


def _make_gather(k0, k1, ch, dtype, nbuf=4):
    kmax = max(k0, k1)
    n_rows = NS * (k0 + k1) * ch
    nbuf = min(nbuf, min(k0, k1))

    @functools.partial(
        pl.kernel, mesh=_mesh,
        out_type=jax.ShapeDtypeStruct((n_rows, H), dtype),
        scratch_types=[
            pltpu.VMEM((kmax, ch), jnp.int32),
            pltpu.VMEM((nbuf, ch, H), dtype),
        ] + [pltpu.SemaphoreType.DMA] * (2 * nbuf),
    )
    def gather_k(idx_hbm, table_hbm, out_hbm, idx_v, bufs, *sems):
        gsem, wsem = sems[:nbuf], sems[nbuf:]
        c = lax.axis_index("c")
        s = lax.axis_index("s")
        wid = c * NS + s
        pltpu.sync_copy(idx_hbm.at[wid], idx_v)

        def ring(k, start_g):
            gd = [None] * k
            wd = [None] * k
            for j in range(nbuf):
                gd[j] = pltpu.async_copy(
                    table_hbm.at[idx_v.at[j]], bufs.at[j], gsem[j])
            for j in range(k):
                b = j % nbuf
                gd[j].wait()
                wd[j] = pltpu.async_copy(
                    bufs.at[b], out_hbm.at[pl.ds((start_g + j) * ch, ch)],
                    wsem[b])
                nj = j + nbuf
                if nj < k:
                    wd[j].wait()
                    gd[nj] = pltpu.async_copy(
                        table_hbm.at[idx_v.at[nj]], bufs.at[b], gsem[b])
            for j in range(max(0, k - nbuf), k):
                wd[j].wait()

        @pl.when(c == 0)
        def _():
            ring(k0, s * k0)

        @pl.when(c == 1)
        def _():
            ring(k1, NS * k0 + s * k1)

    return gather_k


def _split_idx(idx, k0, k1, ch):
    kmax = max(k0, k1)
    flat = idx.reshape(-1, ch)
    c0 = flat[:NS * k0].reshape(NS, k0, ch)
    c0 = jnp.concatenate(
        [c0, jnp.zeros((NS, kmax - k0, ch), jnp.int32)], axis=1) if kmax > k0 else c0
    c1 = flat[NS * k0:].reshape(NS, k1, ch)
    c1 = jnp.concatenate(
        [c1, jnp.zeros((NS, kmax - k1, ch), jnp.int32)], axis=1) if kmax > k1 else c1
    return jnp.concatenate([c0, c1], axis=0)


def _make_scatter(segs, n_seg, sp_rows, nbuf=2):
    ch = segs[0][1]
    assert all(c == ch for _, c in segs)
    chunks = [(si, j) for si, (nc, _) in enumerate(segs) for j in range(nc)]
    n_chunks = len(chunks)
    nbuf = min(nbuf, n_chunks)
    assert sp_rows % (8 * NS) == 0 and n_seg % 8 == 0
    zr = sp_rows // NS
    out_tiles = min(NS, n_seg // 8)
    orows = n_seg // out_tiles

    @functools.partial(
        pl.kernel, mesh=_mesh,
        out_type=(jax.ShapeDtypeStruct((NC, n_seg, H), jnp.float32),
                  jax.ShapeDtypeStruct((NC, n_seg, H), jnp.float32)),
        scratch_types=[
            pltpu.VMEM((len(segs), segs[0][0], ch), jnp.int32),
            pltpu.VMEM((nbuf, ch, H), jnp.float32),
            pltpu.VMEM((nbuf, ch, H), jnp.float32),
            pltpu.VMEM_SHARED((sp_rows, H), jnp.float32),
            pltpu.VMEM_SHARED((sp_rows, H), jnp.float32),
        ] + [pltpu.SemaphoreType.DMA] * (2 * nbuf),
    )
    def scatter_k(*args):
        nin = 3 * len(segs) + 1
        ins, outs, scr = args[:nin], args[nin:nin + 2], args[nin + 2:]
        idx_hbms = ins[0::3][:len(segs)]
        h_hbms = ins[1::3][:len(segs)]
        fc_hbms = ins[2::3][:len(segs)]
        zeros_hbm = ins[-1]
        hs_out, fcs_out = outs
        idx_v, hbuf, fbuf, hsum_sh, fcsum_sh = scr[:5]
        sems = scr[5:]
        hsem, fsem = sems[:nbuf], sems[nbuf:]
        c = lax.axis_index("c")
        s = lax.axis_index("s")
        wid = c * NS + s
        pltpu.sync_copy(zeros_hbm.at[pl.ds(s * zr, zr)], hsum_sh.at[pl.ds(s * zr, zr)])
        pltpu.sync_copy(zeros_hbm.at[pl.ds(s * zr, zr)], fcsum_sh.at[pl.ds(s * zr, zr)])
        for si, (nc, _) in enumerate(segs):
            pltpu.sync_copy(idx_hbms[si].at[wid], idx_v.at[si, pl.ds(0, nc)])
        plsc.subcore_barrier()

        def off(k):
            si, j = chunks[k]
            return si, j, wid * (segs[si][0] * ch) + j * ch

        hd = [None] * n_chunks
        fd = [None] * n_chunks
        for k in range(nbuf):
            si, j, o = off(k)
            hd[k] = pltpu.async_copy(h_hbms[si].at[pl.ds(o, ch)], hbuf.at[k], hsem[k])
            fd[k] = pltpu.async_copy(fc_hbms[si].at[pl.ds(o, ch)], fbuf.at[k], fsem[k])
        for k in range(n_chunks):
            b = k % nbuf
            si, j, o = off(k)
            hd[k].wait()
            fd[k].wait()
            pltpu.sync_copy(hbuf.at[b], hsum_sh.at[idx_v.at[si, j]], add=True)
            pltpu.sync_copy(fbuf.at[b], fcsum_sh.at[idx_v.at[si, j]], add=True)
            nk = k + nbuf
            if nk < n_chunks:
                si2, j2, o2 = off(nk)
                hd[nk] = pltpu.async_copy(h_hbms[si2].at[pl.ds(o2, ch)], hbuf.at[b], hsem[b])
                fd[nk] = pltpu.async_copy(fc_hbms[si2].at[pl.ds(o2, ch)], fbuf.at[b], fsem[b])
        plsc.subcore_barrier()

        @pl.when(s < out_tiles)
        def _():
            pltpu.sync_copy(hsum_sh.at[pl.ds(s * orows, orows)],
                            hs_out.at[c, pl.ds(s * orows, orows)])
            pltpu.sync_copy(fcsum_sh.at[pl.ds(s * orows, orows)],
                            fcs_out.at[c, pl.ds(s * orows, orows)])

    return scatter_k


def _xw_body(x_ref, wiou_ref, biou_ref, wf_ref, bf_ref, xiou_ref, xwf_ref):
    x = x_ref[...]
    xiou_ref[...] = jnp.dot(x, wiou_ref[...], preferred_element_type=jnp.float32) + biou_ref[...]
    xwf_ref[...] = jnp.dot(x, wf_ref[...], preferred_element_type=jnp.float32) + bf_ref[...]


def _cellf3_body(x_ref, wiou_ref, biou_ref, xpf_ref, uf_ref, h_ref, fc_ref):
    iou = jnp.dot(x_ref[...], wiou_ref[...], preferred_element_type=jnp.float32) + biou_ref[...]
    i = jax.nn.sigmoid(iou[:, :H])
    o = jax.nn.sigmoid(iou[:, H:2 * H])
    u = jnp.tanh(iou[:, 2 * H:])
    c = i * u
    h = o * jnp.tanh(c)
    h_ref[...] = h
    pre = xpf_ref[...].astype(jnp.float32) + jnp.dot(
        h, uf_ref[...], preferred_element_type=jnp.float32)
    fc_ref[...] = jax.nn.sigmoid(pre) * c


def _cellf_body(xiou_ref, hs_ref, fcs_ref, uiou_ref, xpf_ref, uf_ref,
                h_ref, fc_ref):
    h_sum = hs_ref[0] + hs_ref[1]
    iou = xiou_ref[...] + jnp.dot(h_sum, uiou_ref[...], preferred_element_type=jnp.float32)
    i = jax.nn.sigmoid(iou[:, :H])
    o = jax.nn.sigmoid(iou[:, H:2 * H])
    u = jnp.tanh(iou[:, 2 * H:])
    c = i * u + (fcs_ref[0] + fcs_ref[1])
    h = o * jnp.tanh(c)
    h_ref[...] = h
    pre = xpf_ref[...].astype(jnp.float32) + jnp.dot(
        h, uf_ref[...], preferred_element_type=jnp.float32)
    fc_ref[...] = jax.nn.sigmoid(pre) * c


def _cell0_body(xiou_ref, hs_ref, fcs_ref, uiou_ref, h_ref):
    h_sum = hs_ref[0] + hs_ref[1]
    iou = xiou_ref[...] + jnp.dot(h_sum, uiou_ref[...], preferred_element_type=jnp.float32)
    i = jax.nn.sigmoid(iou[:, :H])
    o = jax.nn.sigmoid(iou[:, H:2 * H])
    u = jnp.tanh(iou[:, 2 * H:])
    c = i * u + (fcs_ref[0] + fcs_ref[1])
    h_ref[...] = o * jnp.tanh(c)


def _rep(shape):
    return pl.BlockSpec(shape, lambda i: tuple(0 for _ in shape))


def kernel(token_ids, parent_raw, emb, W_iou, U_iou, b_iou, W_f, U_f, b_f):
    f32 = jnp.float32
    token_ids = token_ids.astype(jnp.int32)
    parent_raw = parent_raw.astype(jnp.int32)
    b_iou2 = b_iou.reshape(1, 3 * H)
    b_f2 = b_f.reshape(1, H)

    tid_par = jnp.concatenate([
        token_ids[576:4672], token_ids[64:576], token_ids[:64],
        jnp.zeros((PAR_P - 4672,), jnp.int32)])
    tid_l3 = jnp.concatenate([
        token_ids[4672:], jnp.zeros((P3 - 45328,), jnp.int32)])
    pad2 = P3 - 45328
    par2 = parent_raw[4672:50000] % 4096
    par2_g = jnp.concatenate([L2_OFF + par2, jnp.full((pad2,), 4672, jnp.int32)])
    par2_s = jnp.concatenate([par2, jnp.full((pad2,), 4096, jnp.int32)])
    par1 = parent_raw[576:4672] % 512
    par0 = parent_raw[64:576] % 64
    parg_all = jnp.concatenate([par2_g, L1_OFF + par1, L0_OFF + par0])
    zeros_sp = jnp.zeros((4224, H), f32)

    x_par = _make_gather(3, 1, 80, f32)(_split_idx(tid_par, 3, 1, 80), emb)
    x_l3 = [
        _make_gather(4, 4, 128, f32)(
            _split_idx(tid_l3[p * PC:(p + 1) * PC], 4, 4, 128), emb)
        for p in range(3)]

    xiou_par, xwf_par = pl.pallas_call(
        _xw_body,
        grid=(PAR_P // 256,),
        in_specs=[pl.BlockSpec((256, H), lambda i: (i, 0)),
                  _rep((H, 3 * H)), _rep((1, 3 * H)),
                  _rep((H, H)), _rep((1, H))],
        out_specs=[pl.BlockSpec((256, 3 * H), lambda i: (i, 0)),
                   pl.BlockSpec((256, H), lambda i: (i, 0))],
        out_shape=[jax.ShapeDtypeStruct((PAR_P, 3 * H), f32),
                   jax.ShapeDtypeStruct((PAR_P, H), f32)],
    )(x_par, W_iou, b_iou2, W_f, b_f2)

    xpf_all = _make_gather(24, 6, 112, f32)(
        _split_idx(parg_all, 24, 6, 112), xwf_par)

    h3s, fc2s = [], []
    for p in range(3):
        h3p, fc2p = pl.pallas_call(
            _cellf3_body,
            grid=(PC // 512,),
            in_specs=[pl.BlockSpec((512, H), lambda i: (i, 0)),
                      _rep((H, 3 * H)), _rep((1, 3 * H)),
                      pl.BlockSpec((512, H), lambda i, o=p * (PC // 512): (o + i, 0)),
                      _rep((H, H))],
            out_specs=[pl.BlockSpec((512, H), lambda i: (i, 0))] * 2,
            out_shape=[jax.ShapeDtypeStruct((PC, H), f32)] * 2,
        )(x_l3[p], W_iou, b_iou2, xpf_all, U_f)
        h3s.append(h3p)
        fc2s.append(fc2p)

    def cellf(hs, fcs, n_l, xiou_off, xpf_off):
        blk = min(512, n_l)
        return pl.pallas_call(
            _cellf_body,
            grid=(n_l // blk,),
            in_specs=[pl.BlockSpec((blk, 3 * H), lambda i, o=xiou_off // blk: (o + i, 0)),
                      pl.BlockSpec((NC, blk, H), lambda i: (0, i, 0)),
                      pl.BlockSpec((NC, blk, H), lambda i: (0, i, 0)),
                      _rep((H, 3 * H)),
                      pl.BlockSpec((blk, H), lambda i, o=xpf_off // blk: (o + i, 0)),
                      _rep((H, H))],
            out_specs=[pl.BlockSpec((blk, H), lambda i: (i, 0))] * 2,
            out_shape=[jax.ShapeDtypeStruct((n_l, H), f32)] * 2,
        )(xiou_par, hs, fcs, U_iou, xpf_all, U_f)

    scat2_args = []
    for p in range(3):
        scat2_args += [par2_s[p * PC:(p + 1) * PC].reshape(NW, 8, 64),
                       h3s[p], fc2s[p]]
    hs2, fcs2 = _make_scatter([(8, 64)] * 3, 4096, 4224)(*scat2_args, zeros_sp)
    h2, fc1 = cellf(hs2, fcs2, 4096, L2_OFF, P3)
    hs1, fcs1 = _make_scatter([(1, 128)], 512, 640)(
        par1.reshape(NW, 1, 128), h2, fc1, zeros_sp)
    h1, fc0 = cellf(hs1, fcs1, 512, L1_OFF, P3 + 4096)
    hs0, fcs0 = _make_scatter([(1, 16)], 64, 128)(
        par0.reshape(NW, 1, 16), h1, fc0, zeros_sp)
    h0 = pl.pallas_call(
        _cell0_body,
        grid=(1,),
        in_specs=[pl.BlockSpec((64, 3 * H), lambda i: (L0_OFF // 64 + i, 0)),
                  pl.BlockSpec((NC, 64, H), lambda i: (0, i, 0)),
                  pl.BlockSpec((NC, 64, H), lambda i: (0, i, 0)),
                  _rep((H, 3 * H))],
        out_specs=pl.BlockSpec((64, H), lambda i: (i, 0)),
        out_shape=jax.ShapeDtypeStruct((64, H), f32),
    )(xiou_par, hs0, fcs0, U_iou)
    return h0


# speedup vs baseline: 1.4273x; 1.4273x over previous
import functools

import jax
import jax.numpy as jnp
from jax import lax
from jax.experimental import pallas as pl
from jax.experimental.pallas import tpu as pltpu
from jax.experimental.pallas import tpu_sc as plsc

H = 128
NC, NS = 2, 16
NW = NC * NS

PAR_P = 5120
L2_OFF, L1_OFF, L0_OFF = 0, 4096, 4608
PC = 16384
P3 = 3 * PC
XPF_G = P3 + 4096 + 512

_mesh = plsc.VectorSubcoreMesh(
    core_axis_name="c", subcore_axis_name="s", num_cores=NC, num_subcores=NS)


def _make_gather(k0, k1, ch, dtype, nbuf=4):
    kmax = max(k0, k1)
    n_rows = NS * (k0 + k1) * ch
    nbuf = min(nbuf, min(k0, k1))

    @functools.partial(
        pl.kernel, mesh=_mesh,
        out_type=jax.ShapeDtypeStruct((n_rows, H), dtype),
        scratch_types=[
            pltpu.VMEM((kmax, ch), jnp.int32),
            pltpu.VMEM((nbuf, ch, H), dtype),
        ] + [pltpu.SemaphoreType.DMA] * (2 * nbuf),
    )
    def gather_k(idx_hbm, table_hbm, out_hbm, idx_v, bufs, *sems):
        gsem, wsem = sems[:nbuf], sems[nbuf:]
        c = lax.axis_index("c")
        s = lax.axis_index("s")
        wid = c * NS + s
        pltpu.sync_copy(idx_hbm.at[wid], idx_v)

        def ring(k, start_g):
            gd = [None] * k
            wd = [None] * k
            for j in range(nbuf):
                gd[j] = pltpu.async_copy(
                    table_hbm.at[idx_v.at[j]], bufs.at[j], gsem[j])
            for j in range(k):
                b = j % nbuf
                gd[j].wait()
                wd[j] = pltpu.async_copy(
                    bufs.at[b], out_hbm.at[pl.ds((start_g + j) * ch, ch)],
                    wsem[b])
                nj = j + nbuf
                if nj < k:
                    wd[j].wait()
                    gd[nj] = pltpu.async_copy(
                        table_hbm.at[idx_v.at[nj]], bufs.at[b], gsem[b])
            for j in range(max(0, k - nbuf), k):
                wd[j].wait()

        @pl.when(c == 0)
        def _():
            ring(k0, s * k0)

        @pl.when(c == 1)
        def _():
            ring(k1, NS * k0 + s * k1)

    return gather_k


def _make_spmem_gather(k0, k1, ch, trows, nbuf=4):
    kmax = max(k0, k1)
    n_rows = NS * (k0 + k1) * ch
    nbuf = min(nbuf, min(k0, k1))
    assert trows % (8 * NS) == 0
    tr = trows // NS

    @functools.partial(
        pl.kernel, mesh=_mesh,
        out_type=jax.ShapeDtypeStruct((n_rows, H), jnp.float32),
        scratch_types=[
            pltpu.VMEM((kmax, ch), jnp.int32),
            pltpu.VMEM((nbuf, ch, H), jnp.float32),
            pltpu.VMEM_SHARED((trows, H), jnp.float32),
        ] + [pltpu.SemaphoreType.DMA] * (2 * nbuf),
    )
    def gather_k(idx_hbm, table_hbm, out_hbm, idx_v, bufs, tb_sh, *sems):
        gsem, wsem = sems[:nbuf], sems[nbuf:]
        c = lax.axis_index("c")
        s = lax.axis_index("s")
        wid = c * NS + s
        pltpu.sync_copy(table_hbm.at[pl.ds(s * tr, tr)], tb_sh.at[pl.ds(s * tr, tr)])
        pltpu.sync_copy(idx_hbm.at[wid], idx_v)
        plsc.subcore_barrier()

        def ring(k, start_g):
            gd = [None] * k
            wd = [None] * k
            for j in range(nbuf):
                gd[j] = pltpu.async_copy(
                    tb_sh.at[idx_v.at[j]], bufs.at[j], gsem[j])
            for j in range(k):
                b = j % nbuf
                gd[j].wait()
                wd[j] = pltpu.async_copy(
                    bufs.at[b], out_hbm.at[pl.ds((start_g + j) * ch, ch)],
                    wsem[b])
                nj = j + nbuf
                if nj < k:
                    wd[j].wait()
                    gd[nj] = pltpu.async_copy(
                        tb_sh.at[idx_v.at[nj]], bufs.at[b], gsem[b])
            for j in range(max(0, k - nbuf), k):
                wd[j].wait()

        @pl.when(c == 0)
        def _():
            ring(k0, s * k0)

        @pl.when(c == 1)
        def _():
            ring(k1, NS * k0 + s * k1)

    return gather_k


def _split_idx(idx, k0, k1, ch):
    kmax = max(k0, k1)
    flat = idx.reshape(-1, ch)
    c0 = flat[:NS * k0].reshape(NS, k0, ch)
    c0 = jnp.concatenate(
        [c0, jnp.zeros((NS, kmax - k0, ch), jnp.int32)], axis=1) if kmax > k0 else c0
    c1 = flat[NS * k0:].reshape(NS, k1, ch)
    c1 = jnp.concatenate(
        [c1, jnp.zeros((NS, kmax - k1, ch), jnp.int32)], axis=1) if kmax > k1 else c1
    return jnp.concatenate([c0, c1], axis=0)


def _make_scatter(segs, n_seg, sp_rows, nbuf=2):
    ch = segs[0][1]
    assert all(c == ch for _, c in segs)
    chunks = [(si, j) for si, (nc, _) in enumerate(segs) for j in range(nc)]
    n_chunks = len(chunks)
    nbuf = min(nbuf, n_chunks)
    assert sp_rows % (8 * NS) == 0 and n_seg % 8 == 0
    zr = sp_rows // NS
    out_tiles = min(NS, n_seg // 8)
    orows = n_seg // out_tiles

    @functools.partial(
        pl.kernel, mesh=_mesh,
        out_type=(jax.ShapeDtypeStruct((NC, n_seg, H), jnp.float32),
                  jax.ShapeDtypeStruct((NC, n_seg, H), jnp.float32)),
        scratch_types=[
            pltpu.VMEM((len(segs), segs[0][0], ch), jnp.int32),
            pltpu.VMEM((nbuf, ch, H), jnp.float32),
            pltpu.VMEM((nbuf, ch, H), jnp.float32),
            pltpu.VMEM_SHARED((sp_rows, H), jnp.float32),
            pltpu.VMEM_SHARED((sp_rows, H), jnp.float32),
        ] + [pltpu.SemaphoreType.DMA] * (2 * nbuf),
    )
    def scatter_k(*args):
        nin = 3 * len(segs) + 1
        ins, outs, scr = args[:nin], args[nin:nin + 2], args[nin + 2:]
        idx_hbms = ins[0::3][:len(segs)]
        h_hbms = ins[1::3][:len(segs)]
        fc_hbms = ins[2::3][:len(segs)]
        zeros_hbm = ins[-1]
        hs_out, fcs_out = outs
        idx_v, hbuf, fbuf, hsum_sh, fcsum_sh = scr[:5]
        sems = scr[5:]
        hsem, fsem = sems[:nbuf], sems[nbuf:]
        c = lax.axis_index("c")
        s = lax.axis_index("s")
        wid = c * NS + s
        pltpu.sync_copy(zeros_hbm.at[pl.ds(s * zr, zr)], hsum_sh.at[pl.ds(s * zr, zr)])
        pltpu.sync_copy(zeros_hbm.at[pl.ds(s * zr, zr)], fcsum_sh.at[pl.ds(s * zr, zr)])
        for si, (nc, _) in enumerate(segs):
            pltpu.sync_copy(idx_hbms[si].at[wid], idx_v.at[si, pl.ds(0, nc)])
        plsc.subcore_barrier()

        def off(k):
            si, j = chunks[k]
            return si, j, wid * (segs[si][0] * ch) + j * ch

        hd = [None] * n_chunks
        fd = [None] * n_chunks
        for k in range(nbuf):
            si, j, o = off(k)
            hd[k] = pltpu.async_copy(h_hbms[si].at[pl.ds(o, ch)], hbuf.at[k], hsem[k])
            fd[k] = pltpu.async_copy(fc_hbms[si].at[pl.ds(o, ch)], fbuf.at[k], fsem[k])
        for k in range(n_chunks):
            b = k % nbuf
            si, j, o = off(k)
            hd[k].wait()
            fd[k].wait()
            pltpu.sync_copy(hbuf.at[b], hsum_sh.at[idx_v.at[si, j]], add=True)
            pltpu.sync_copy(fbuf.at[b], fcsum_sh.at[idx_v.at[si, j]], add=True)
            nk = k + nbuf
            if nk < n_chunks:
                si2, j2, o2 = off(nk)
                hd[nk] = pltpu.async_copy(h_hbms[si2].at[pl.ds(o2, ch)], hbuf.at[b], hsem[b])
                fd[nk] = pltpu.async_copy(fc_hbms[si2].at[pl.ds(o2, ch)], fbuf.at[b], fsem[b])
        plsc.subcore_barrier()

        @pl.when(s < out_tiles)
        def _():
            pltpu.sync_copy(hsum_sh.at[pl.ds(s * orows, orows)],
                            hs_out.at[c, pl.ds(s * orows, orows)])
            pltpu.sync_copy(fcsum_sh.at[pl.ds(s * orows, orows)],
                            fcs_out.at[c, pl.ds(s * orows, orows)])

    return scatter_k


def _xw_body(x_ref, wiou_ref, biou_ref, wf_ref, bf_ref, xiou_ref, xwf_ref):
    x = x_ref[...]
    xiou_ref[...] = jnp.dot(x, wiou_ref[...], preferred_element_type=jnp.float32) + biou_ref[...]
    xwf_ref[...] = jnp.dot(x, wf_ref[...], preferred_element_type=jnp.float32) + bf_ref[...]


def _cellf3_body(x_ref, wiou_ref, biou_ref, xpf_ref, uf_ref, h_ref, fc_ref):
    iou = jnp.dot(x_ref[...], wiou_ref[...], preferred_element_type=jnp.float32) + biou_ref[...]
    i = jax.nn.sigmoid(iou[:, :H])
    o = jax.nn.sigmoid(iou[:, H:2 * H])
    u = jnp.tanh(iou[:, 2 * H:])
    c = i * u
    h = o * jnp.tanh(c)
    h_ref[...] = h
    pre = xpf_ref[...].astype(jnp.float32) + jnp.dot(
        h, uf_ref[...], preferred_element_type=jnp.float32)
    fc_ref[...] = jax.nn.sigmoid(pre) * c


def _cellf_body(xiou_ref, hs_ref, fcs_ref, uiou_ref, xpf_ref, uf_ref,
                h_ref, fc_ref):
    h_sum = hs_ref[0] + hs_ref[1]
    iou = xiou_ref[...] + jnp.dot(h_sum, uiou_ref[...], preferred_element_type=jnp.float32)
    i = jax.nn.sigmoid(iou[:, :H])
    o = jax.nn.sigmoid(iou[:, H:2 * H])
    u = jnp.tanh(iou[:, 2 * H:])
    c = i * u + (fcs_ref[0] + fcs_ref[1])
    h = o * jnp.tanh(c)
    h_ref[...] = h
    pre = xpf_ref[...].astype(jnp.float32) + jnp.dot(
        h, uf_ref[...], preferred_element_type=jnp.float32)
    fc_ref[...] = jax.nn.sigmoid(pre) * c


def _cell0_body(xiou_ref, hs_ref, fcs_ref, uiou_ref, h_ref):
    h_sum = hs_ref[0] + hs_ref[1]
    iou = xiou_ref[...] + jnp.dot(h_sum, uiou_ref[...], preferred_element_type=jnp.float32)
    i = jax.nn.sigmoid(iou[:, :H])
    o = jax.nn.sigmoid(iou[:, H:2 * H])
    u = jnp.tanh(iou[:, 2 * H:])
    c = i * u + (fcs_ref[0] + fcs_ref[1])
    h_ref[...] = o * jnp.tanh(c)


def _rep(shape):
    return pl.BlockSpec(shape, lambda i: tuple(0 for _ in shape))


def kernel(token_ids, parent_raw, emb, W_iou, U_iou, b_iou, W_f, U_f, b_f):
    f32 = jnp.float32
    token_ids = token_ids.astype(jnp.int32)
    parent_raw = parent_raw.astype(jnp.int32)
    b_iou2 = b_iou.reshape(1, 3 * H)
    b_f2 = b_f.reshape(1, H)

    tid_par = jnp.concatenate([
        token_ids[576:4672], token_ids[64:576], token_ids[:64],
        jnp.zeros((PAR_P - 4672,), jnp.int32)])
    tid_l3 = jnp.concatenate([
        token_ids[4672:], jnp.zeros((P3 - 45328,), jnp.int32)])
    pad2 = P3 - 45328
    par2 = parent_raw[4672:50000] % 4096
    par2_g = jnp.concatenate([L2_OFF + par2, jnp.full((pad2,), 4672, jnp.int32)])
    par2_s = jnp.concatenate([par2, jnp.full((pad2,), 4096, jnp.int32)])
    par1 = parent_raw[576:4672] % 512
    par0 = parent_raw[64:576] % 64
    parg_all = jnp.concatenate([par2_g, L1_OFF + par1, L0_OFF + par0])
    zeros_sp = jnp.zeros((4224, H), f32)

    def _dep(arr):
        return (arr[0, 0] * 0).astype(jnp.int32)

    x_par = _make_gather(3, 1, 80, f32)(_split_idx(tid_par, 3, 1, 80), emb)
    x_l3a = _make_gather(4, 4, 128, f32)(
        _split_idx(tid_l3[:PC] + _dep(x_par), 4, 4, 128), emb)

    xiou_par, xwf_par = pl.pallas_call(
        _xw_body,
        grid=(PAR_P // 256,),
        in_specs=[pl.BlockSpec((256, H), lambda i: (i, 0)),
                  _rep((H, 3 * H)), _rep((1, 3 * H)),
                  _rep((H, H)), _rep((1, H))],
        out_specs=[pl.BlockSpec((256, 3 * H), lambda i: (i, 0)),
                   pl.BlockSpec((256, H), lambda i: (i, 0))],
        out_shape=[jax.ShapeDtypeStruct((PAR_P, 3 * H), f32),
                   jax.ShapeDtypeStruct((PAR_P, H), f32)],
    )(x_par, W_iou, b_iou2, W_f, b_f2)

    xpf_all = _make_spmem_gather(15, 15, 112, PAR_P)(
        _split_idx(parg_all + _dep(x_l3a), 15, 15, 112), xwf_par)
    x_l3b = _make_gather(4, 4, 128, f32)(
        _split_idx(tid_l3[PC:2 * PC] + _dep(xpf_all), 4, 4, 128), emb)
    x_l3c = _make_gather(4, 4, 128, f32)(
        _split_idx(tid_l3[2 * PC:] + _dep(x_l3b), 4, 4, 128), emb)
    x_l3 = [x_l3a, x_l3b, x_l3c]

    h3s, fc2s = [], []
    for p in range(3):
        h3p, fc2p = pl.pallas_call(
            _cellf3_body,
            grid=(PC // 512,),
            in_specs=[pl.BlockSpec((512, H), lambda i: (i, 0)),
                      _rep((H, 3 * H)), _rep((1, 3 * H)),
                      pl.BlockSpec((512, H), lambda i, o=p * (PC // 512): (o + i, 0)),
                      _rep((H, H))],
            out_specs=[pl.BlockSpec((512, H), lambda i: (i, 0))] * 2,
            out_shape=[jax.ShapeDtypeStruct((PC, H), f32)] * 2,
        )(x_l3[p], W_iou, b_iou2, xpf_all, U_f)
        h3s.append(h3p)
        fc2s.append(fc2p)

    def cellf(hs, fcs, n_l, xiou_off, xpf_off):
        blk = min(512, n_l)
        return pl.pallas_call(
            _cellf_body,
            grid=(n_l // blk,),
            in_specs=[pl.BlockSpec((blk, 3 * H), lambda i, o=xiou_off // blk: (o + i, 0)),
                      pl.BlockSpec((NC, blk, H), lambda i: (0, i, 0)),
                      pl.BlockSpec((NC, blk, H), lambda i: (0, i, 0)),
                      _rep((H, 3 * H)),
                      pl.BlockSpec((blk, H), lambda i, o=xpf_off // blk: (o + i, 0)),
                      _rep((H, H))],
            out_specs=[pl.BlockSpec((blk, H), lambda i: (i, 0))] * 2,
            out_shape=[jax.ShapeDtypeStruct((n_l, H), f32)] * 2,
        )(xiou_par, hs, fcs, U_iou, xpf_all, U_f)

    scat2_args = []
    for p in range(3):
        scat2_args += [par2_s[p * PC:(p + 1) * PC].reshape(NW, 8, 64),
                       h3s[p], fc2s[p]]
    hs2, fcs2 = _make_scatter([(8, 64)] * 3, 4096, 4224)(*scat2_args, zeros_sp)
    h2, fc1 = cellf(hs2, fcs2, 4096, L2_OFF, P3)
    hs1, fcs1 = _make_scatter([(1, 128)], 512, 640)(
        par1.reshape(NW, 1, 128), h2, fc1, zeros_sp)
    h1, fc0 = cellf(hs1, fcs1, 512, L1_OFF, P3 + 4096)
    hs0, fcs0 = _make_scatter([(1, 16)], 64, 128)(
        par0.reshape(NW, 1, 16), h1, fc0, zeros_sp)
    h0 = pl.pallas_call(
        _cell0_body,
        grid=(1,),
        in_specs=[pl.BlockSpec((64, 3 * H), lambda i: (L0_OFF // 64 + i, 0)),
                  pl.BlockSpec((NC, 64, H), lambda i: (0, i, 0)),
                  pl.BlockSpec((NC, 64, H), lambda i: (0, i, 0)),
                  _rep((H, 3 * H))],
        out_specs=pl.BlockSpec((64, H), lambda i: (i, 0)),
        out_shape=jax.ShapeDtypeStruct((64, H), f32),
    )(xiou_par, hs0, fcs0, U_iou)
    return h0


# speedup vs baseline: 1.4662x; 1.0273x over previous
import functools

import jax
import jax.numpy as jnp
from jax import lax
from jax.experimental import pallas as pl
from jax.experimental.pallas import tpu as pltpu
from jax.experimental.pallas import tpu_sc as plsc

H = 128
NC, NS = 2, 16
NW = NC * NS

PAR_P = 5120
L2_OFF, L1_OFF, L0_OFF = 0, 4096, 4608
PC = 16384
P3 = 3 * PC
XPF_G = P3 + 4096 + 512

_mesh = plsc.VectorSubcoreMesh(
    core_axis_name="c", subcore_axis_name="s", num_cores=NC, num_subcores=NS)


def _make_gather(k0, k1, ch, dtype, nbuf=4):
    kmax = max(k0, k1)
    n_rows = NS * (k0 + k1) * ch
    nbuf = min(nbuf, min(k0, k1))

    @functools.partial(
        pl.kernel, mesh=_mesh,
        out_type=jax.ShapeDtypeStruct((n_rows, H), dtype),
        scratch_types=[
            pltpu.VMEM((kmax, ch), jnp.int32),
            pltpu.VMEM((nbuf, ch, H), dtype),
        ] + [pltpu.SemaphoreType.DMA] * (2 * nbuf),
    )
    def gather_k(idx_hbm, table_hbm, out_hbm, idx_v, bufs, *sems):
        gsem, wsem = sems[:nbuf], sems[nbuf:]
        c = lax.axis_index("c")
        s = lax.axis_index("s")
        wid = c * NS + s
        pltpu.sync_copy(idx_hbm.at[wid], idx_v)

        def ring(k, start_g):
            gd = [None] * k
            wd = [None] * k
            for j in range(nbuf):
                gd[j] = pltpu.async_copy(
                    table_hbm.at[idx_v.at[j]], bufs.at[j], gsem[j])
            for j in range(k):
                b = j % nbuf
                gd[j].wait()
                wd[j] = pltpu.async_copy(
                    bufs.at[b], out_hbm.at[pl.ds((start_g + j) * ch, ch)],
                    wsem[b])
                nj = j + nbuf
                if nj < k:
                    wd[j].wait()
                    gd[nj] = pltpu.async_copy(
                        table_hbm.at[idx_v.at[nj]], bufs.at[b], gsem[b])
            for j in range(max(0, k - nbuf), k):
                wd[j].wait()

        @pl.when(c == 0)
        def _():
            ring(k0, s * k0)

        @pl.when(c == 1)
        def _():
            ring(k1, NS * k0 + s * k1)

    return gather_k


def _make_spmem_gather(k0, k1, ch, trows, nbuf=4):
    kmax = max(k0, k1)
    n_rows = NS * (k0 + k1) * ch
    nbuf = min(nbuf, min(k0, k1))
    assert trows % (8 * NS) == 0
    tr = trows // NS

    @functools.partial(
        pl.kernel, mesh=_mesh,
        out_type=jax.ShapeDtypeStruct((n_rows, H), jnp.float32),
        scratch_types=[
            pltpu.VMEM((kmax, ch), jnp.int32),
            pltpu.VMEM((nbuf, ch, H), jnp.float32),
            pltpu.VMEM_SHARED((trows, H), jnp.float32),
        ] + [pltpu.SemaphoreType.DMA] * (2 * nbuf),
    )
    def gather_k(idx_hbm, table_hbm, out_hbm, idx_v, bufs, tb_sh, *sems):
        gsem, wsem = sems[:nbuf], sems[nbuf:]
        c = lax.axis_index("c")
        s = lax.axis_index("s")
        wid = c * NS + s
        pltpu.sync_copy(table_hbm.at[pl.ds(s * tr, tr)], tb_sh.at[pl.ds(s * tr, tr)])
        pltpu.sync_copy(idx_hbm.at[wid], idx_v)
        plsc.subcore_barrier()

        def ring(k, start_g):
            gd = [None] * k
            wd = [None] * k
            for j in range(nbuf):
                gd[j] = pltpu.async_copy(
                    tb_sh.at[idx_v.at[j]], bufs.at[j], gsem[j])
            for j in range(k):
                b = j % nbuf
                gd[j].wait()
                wd[j] = pltpu.async_copy(
                    bufs.at[b], out_hbm.at[pl.ds((start_g + j) * ch, ch)],
                    wsem[b])
                nj = j + nbuf
                if nj < k:
                    wd[j].wait()
                    gd[nj] = pltpu.async_copy(
                        tb_sh.at[idx_v.at[nj]], bufs.at[b], gsem[b])
            for j in range(max(0, k - nbuf), k):
                wd[j].wait()

        @pl.when(c == 0)
        def _():
            ring(k0, s * k0)

        @pl.when(c == 1)
        def _():
            ring(k1, NS * k0 + s * k1)

    return gather_k


def _split_idx(idx, k0, k1, ch):
    kmax = max(k0, k1)
    flat = idx.reshape(-1, ch)
    c0 = flat[:NS * k0].reshape(NS, k0, ch)
    c0 = jnp.concatenate(
        [c0, jnp.zeros((NS, kmax - k0, ch), jnp.int32)], axis=1) if kmax > k0 else c0
    c1 = flat[NS * k0:].reshape(NS, k1, ch)
    c1 = jnp.concatenate(
        [c1, jnp.zeros((NS, kmax - k1, ch), jnp.int32)], axis=1) if kmax > k1 else c1
    return jnp.concatenate([c0, c1], axis=0)


def _make_scatter(segs, n_seg, sp_rows, nbuf=2):
    ch = segs[0][1]
    assert all(c == ch for _, c in segs)
    chunks = [(si, j) for si, (nc, _) in enumerate(segs) for j in range(nc)]
    n_chunks = len(chunks)
    nbuf = min(nbuf, n_chunks)
    assert sp_rows % (8 * NS) == 0 and n_seg % 8 == 0
    out_tiles = min(NS, n_seg // 8)
    orows = n_seg // out_tiles

    @functools.partial(
        pl.kernel, mesh=_mesh,
        out_type=(jax.ShapeDtypeStruct((NC, n_seg, H), jnp.float32),
                  jax.ShapeDtypeStruct((NC, n_seg, H), jnp.float32)),
        scratch_types=[
            pltpu.VMEM((len(segs), segs[0][0], ch), jnp.int32),
            pltpu.VMEM((nbuf, ch, H), jnp.float32),
            pltpu.VMEM((nbuf, ch, H), jnp.float32),
            pltpu.VMEM_SHARED((sp_rows, H), jnp.float32),
            pltpu.VMEM_SHARED((sp_rows, H), jnp.float32),
        ] + [pltpu.SemaphoreType.DMA] * (2 * nbuf),
    )
    def scatter_k(*args):
        nin = 3 * len(segs) + 2
        ins, outs, scr = args[:nin], args[nin:nin + 2], args[nin + 2:]
        idx_hbms = ins[0::3][:len(segs)]
        h_hbms = ins[1::3][:len(segs)]
        fc_hbms = ins[2::3][:len(segs)]
        inith_hbm, initf_hbm = ins[-2], ins[-1]
        hs_out, fcs_out = outs
        idx_v, hbuf, fbuf, hsum_sh, fcsum_sh = scr[:5]
        sems = scr[5:]
        hsem, fsem = sems[:nbuf], sems[nbuf:]
        c = lax.axis_index("c")
        s = lax.axis_index("s")
        wid = c * NS + s
        @pl.when(s < out_tiles)
        def _():
            pltpu.sync_copy(inith_hbm.at[c, pl.ds(s * orows, orows)],
                            hsum_sh.at[pl.ds(s * orows, orows)])
            pltpu.sync_copy(initf_hbm.at[c, pl.ds(s * orows, orows)],
                            fcsum_sh.at[pl.ds(s * orows, orows)])
        for si, (nc, _) in enumerate(segs):
            pltpu.sync_copy(idx_hbms[si].at[wid], idx_v.at[si, pl.ds(0, nc)])
        plsc.subcore_barrier()

        def off(k):
            si, j = chunks[k]
            return si, j, wid * (segs[si][0] * ch) + j * ch

        hd = [None] * n_chunks
        fd = [None] * n_chunks
        for k in range(nbuf):
            si, j, o = off(k)
            hd[k] = pltpu.async_copy(h_hbms[si].at[pl.ds(o, ch)], hbuf.at[k], hsem[k])
            fd[k] = pltpu.async_copy(fc_hbms[si].at[pl.ds(o, ch)], fbuf.at[k], fsem[k])
        for k in range(n_chunks):
            b = k % nbuf
            si, j, o = off(k)
            hd[k].wait()
            fd[k].wait()
            pltpu.sync_copy(hbuf.at[b], hsum_sh.at[idx_v.at[si, j]], add=True)
            pltpu.sync_copy(fbuf.at[b], fcsum_sh.at[idx_v.at[si, j]], add=True)
            nk = k + nbuf
            if nk < n_chunks:
                si2, j2, o2 = off(nk)
                hd[nk] = pltpu.async_copy(h_hbms[si2].at[pl.ds(o2, ch)], hbuf.at[b], hsem[b])
                fd[nk] = pltpu.async_copy(fc_hbms[si2].at[pl.ds(o2, ch)], fbuf.at[b], fsem[b])
        plsc.subcore_barrier()

        @pl.when(s < out_tiles)
        def _():
            pltpu.sync_copy(hsum_sh.at[pl.ds(s * orows, orows)],
                            hs_out.at[c, pl.ds(s * orows, orows)])
            pltpu.sync_copy(fcsum_sh.at[pl.ds(s * orows, orows)],
                            fcs_out.at[c, pl.ds(s * orows, orows)])

    return scatter_k


def _xw_body(x_ref, wiou_ref, biou_ref, wf_ref, bf_ref, xiou_ref, xwf_ref):
    x = x_ref[...]
    xiou_ref[...] = jnp.dot(x, wiou_ref[...], preferred_element_type=jnp.float32) + biou_ref[...]
    xwf_ref[...] = jnp.dot(x, wf_ref[...], preferred_element_type=jnp.float32) + bf_ref[...]


def _cellf3_body(x_ref, wiou_ref, biou_ref, xpf_ref, uf_ref, h_ref, fc_ref):
    iou = jnp.dot(x_ref[...], wiou_ref[...], preferred_element_type=jnp.float32) + biou_ref[...]
    i = jax.nn.sigmoid(iou[:, :H])
    o = jax.nn.sigmoid(iou[:, H:2 * H])
    u = jnp.tanh(iou[:, 2 * H:])
    c = i * u
    h = o * jnp.tanh(c)
    h_ref[...] = h
    pre = xpf_ref[...].astype(jnp.float32) + jnp.dot(
        h, uf_ref[...], preferred_element_type=jnp.float32)
    fc_ref[...] = jax.nn.sigmoid(pre) * c


def _cellf_body(xiou_ref, hs_ref, fcs_ref, uiou_ref, xpf_ref, uf_ref,
                h_ref, fc_ref):
    h_sum = hs_ref[0] + hs_ref[1]
    iou = xiou_ref[...] + jnp.dot(h_sum, uiou_ref[...], preferred_element_type=jnp.float32)
    i = jax.nn.sigmoid(iou[:, :H])
    o = jax.nn.sigmoid(iou[:, H:2 * H])
    u = jnp.tanh(iou[:, 2 * H:])
    c = i * u + (fcs_ref[0] + fcs_ref[1])
    h = o * jnp.tanh(c)
    h_ref[...] = h
    pre = xpf_ref[...].astype(jnp.float32) + jnp.dot(
        h, uf_ref[...], preferred_element_type=jnp.float32)
    fc_ref[...] = jax.nn.sigmoid(pre) * c


def _cell0_body(xiou_ref, hs_ref, fcs_ref, uiou_ref, h_ref):
    h_sum = hs_ref[0] + hs_ref[1]
    iou = xiou_ref[...] + jnp.dot(h_sum, uiou_ref[...], preferred_element_type=jnp.float32)
    i = jax.nn.sigmoid(iou[:, :H])
    o = jax.nn.sigmoid(iou[:, H:2 * H])
    u = jnp.tanh(iou[:, 2 * H:])
    c = i * u + (fcs_ref[0] + fcs_ref[1])
    h_ref[...] = o * jnp.tanh(c)


def _rep(shape):
    return pl.BlockSpec(shape, lambda i: tuple(0 for _ in shape))


def kernel(token_ids, parent_raw, emb, W_iou, U_iou, b_iou, W_f, U_f, b_f):
    f32 = jnp.float32
    token_ids = token_ids.astype(jnp.int32)
    parent_raw = parent_raw.astype(jnp.int32)
    b_iou2 = b_iou.reshape(1, 3 * H)
    b_f2 = b_f.reshape(1, H)

    tid_par = jnp.concatenate([
        token_ids[576:4672], token_ids[64:576], token_ids[:64],
        jnp.zeros((PAR_P - 4672,), jnp.int32)])
    tid_l3 = jnp.concatenate([
        token_ids[4672:], jnp.zeros((P3 - 45328,), jnp.int32)])
    pad2 = P3 - 45328
    par2 = parent_raw[4672:50000] % 4096
    par2_g = jnp.concatenate([L2_OFF + par2, jnp.full((pad2,), 4672, jnp.int32)])
    par2_s = jnp.concatenate([par2, jnp.full((pad2,), 4096, jnp.int32)])
    par1 = parent_raw[576:4672] % 512
    par0 = parent_raw[64:576] % 64
    parg_all = jnp.concatenate([par2_g, L1_OFF + par1, L0_OFF + par0])

    def _dep(arr):
        return (arr[0, 0] * 0).astype(jnp.int32)

    x_par = _make_gather(3, 1, 80, f32)(_split_idx(tid_par, 3, 1, 80), emb)
    x_l3a = _make_gather(4, 4, 128, f32)(
        _split_idx(tid_l3[:PC] + _dep(x_par), 4, 4, 128), emb)

    xiou_par, xwf_par = pl.pallas_call(
        _xw_body,
        grid=(PAR_P // 256,),
        in_specs=[pl.BlockSpec((256, H), lambda i: (i, 0)),
                  _rep((H, 3 * H)), _rep((1, 3 * H)),
                  _rep((H, H)), _rep((1, H))],
        out_specs=[pl.BlockSpec((256, 3 * H), lambda i: (i, 0)),
                   pl.BlockSpec((256, H), lambda i: (i, 0))],
        out_shape=[jax.ShapeDtypeStruct((PAR_P, 3 * H), f32),
                   jax.ShapeDtypeStruct((PAR_P, H), f32)],
    )(x_par, W_iou, b_iou2, W_f, b_f2)

    xpf_all = _make_spmem_gather(15, 15, 112, PAR_P)(
        _split_idx(parg_all + _dep(x_l3a), 15, 15, 112), xwf_par)
    x_l3b = _make_gather(4, 4, 128, f32)(
        _split_idx(tid_l3[PC:2 * PC] + _dep(xpf_all), 4, 4, 128), emb)
    x_l3c = _make_gather(4, 4, 128, f32)(
        _split_idx(tid_l3[2 * PC:] + _dep(x_l3b), 4, 4, 128), emb)
    x_l3 = [x_l3a, x_l3b, x_l3c]

    h3s, fc2s = [], []
    for p in range(3):
        h3p, fc2p = pl.pallas_call(
            _cellf3_body,
            grid=(PC // 512,),
            in_specs=[pl.BlockSpec((512, H), lambda i: (i, 0)),
                      _rep((H, 3 * H)), _rep((1, 3 * H)),
                      pl.BlockSpec((512, H), lambda i, o=p * (PC // 512): (o + i, 0)),
                      _rep((H, H))],
            out_specs=[pl.BlockSpec((512, H), lambda i: (i, 0))] * 2,
            out_shape=[jax.ShapeDtypeStruct((PC, H), f32)] * 2,
        )(x_l3[p], W_iou, b_iou2, xpf_all, U_f)
        h3s.append(h3p)
        fc2s.append(fc2p)

    def cellf(hs, fcs, n_l, xiou_off, xpf_off):
        blk = min(512, n_l)
        return pl.pallas_call(
            _cellf_body,
            grid=(n_l // blk,),
            in_specs=[pl.BlockSpec((blk, 3 * H), lambda i, o=xiou_off // blk: (o + i, 0)),
                      pl.BlockSpec((NC, blk, H), lambda i: (0, i, 0)),
                      pl.BlockSpec((NC, blk, H), lambda i: (0, i, 0)),
                      _rep((H, 3 * H)),
                      pl.BlockSpec((blk, H), lambda i, o=xpf_off // blk: (o + i, 0)),
                      _rep((H, H))],
            out_specs=[pl.BlockSpec((blk, H), lambda i: (i, 0))] * 2,
            out_shape=[jax.ShapeDtypeStruct((n_l, H), f32)] * 2,
        )(xiou_par, hs, fcs, U_iou, xpf_all, U_f)

    z2 = jnp.zeros((NC, 4096, H), f32)
    seg = lambda p: [par2_s[p * PC:(p + 1) * PC].reshape(NW, 8, 64),
                     h3s[p], fc2s[p]]
    hs2ab, fcs2ab = _make_scatter([(8, 64)] * 2, 4096, 4224)(
        *seg(0), *seg(1), z2, z2)
    hs2, fcs2 = _make_scatter([(8, 64)], 4096, 4224)(
        *seg(2), hs2ab, fcs2ab)
    h2, fc1 = cellf(hs2, fcs2, 4096, L2_OFF, P3)
    z1 = jnp.zeros((NC, 512, H), f32)
    hs1, fcs1 = _make_scatter([(1, 128)], 512, 640)(
        par1.reshape(NW, 1, 128), h2, fc1, z1, z1)
    h1, fc0 = cellf(hs1, fcs1, 512, L1_OFF, P3 + 4096)
    z0 = jnp.zeros((NC, 64, H), f32)
    hs0, fcs0 = _make_scatter([(1, 16)], 64, 128)(
        par0.reshape(NW, 1, 16), h1, fc0, z0, z0)
    h0 = pl.pallas_call(
        _cell0_body,
        grid=(1,),
        in_specs=[pl.BlockSpec((64, 3 * H), lambda i: (L0_OFF // 64 + i, 0)),
                  pl.BlockSpec((NC, 64, H), lambda i: (0, i, 0)),
                  pl.BlockSpec((NC, 64, H), lambda i: (0, i, 0)),
                  _rep((H, 3 * H))],
        out_specs=pl.BlockSpec((64, H), lambda i: (i, 0)),
        out_shape=jax.ShapeDtypeStruct((64, H), f32),
    )(xiou_par, hs0, fcs0, U_iou)
    return h0
